# baseline (speedup 1.0000x reference)
import math

import jax
import jax.numpy as jnp
from jax import lax
from jax.experimental import pallas as pl
from jax.experimental.pallas import tpu as pltpu

_N = 32768
_B = 16
_D = 128
_DP = 64
_H = 4
_BN = 4096
_G = _N // _BN
_SCALE = 1.0 / math.sqrt(float(_DP))


def _body(x_ref, seg_ref, wk_ref, wq_ref, out_ref, e_ref, stats_ref, u_ref):
    p = pl.program_id(0)
    g = pl.program_id(1)
    seg = jnp.broadcast_to(seg_ref[...], (_H, _BN))

    @pl.when((p == 0) & (g == 0))
    def _fold_u():
        wqt = jnp.transpose(wq_ref[...])
        tiled = jnp.concatenate([wqt] * _H, axis=0)
        r = lax.broadcasted_iota(jnp.int32, (_H * _DP, _H), 0)
        c = lax.broadcasted_iota(jnp.int32, (_H * _DP, _H), 1)
        wqbd = jnp.where(r // _DP == c, tiled, 0.0)
        u_ref[...] = lax.dot_general(wk_ref[...], wqbd,
                                     (((1,), (0,)), ((), ())),
                                     precision=lax.Precision.HIGHEST) * _SCALE

    @pl.when(p == 0)
    def _phase0():
        t = lax.dot_general(x_ref[...], u_ref[...], (((1,), (0,)), ((), ())))
        e = jnp.exp(jnp.minimum(jnp.transpose(t), 50.0))
        e_ref[:, pl.ds(g * _BN, _BN)] = e
        out_ref[...] = e
        ohT = (lax.broadcasted_iota(jnp.int32, (_B, _BN), 0)
               == seg_ref[...]).astype(jnp.float32)
        part = jnp.transpose(
            lax.dot_general(ohT, e, (((1,), (1,)), ((), ()))))

        @pl.when(g == 0)
        def _init():
            stats_ref[...] = part

        @pl.when(g != 0)
        def _acc():
            stats_ref[...] = stats_ref[...] + part

    @pl.when(p == 1)
    def _phase1():
        e = e_ref[:, pl.ds(g * _BN, _BN)]
        recip = 1.0 / jnp.maximum(stats_ref[...], 1e-30)
        ohT = (lax.broadcasted_iota(jnp.int32, (_B, _BN), 0)
               == seg_ref[...]).astype(jnp.float32)
        rg = lax.dot_general(recip, ohT, (((1,), (0,)), ((), ())),
                             precision=lax.Precision.HIGHEST)
        out_ref[...] = e * rg


def _make_call(interpret=False):
    return pl.pallas_call(
        _body,
        grid=(2, _G),
        in_specs=[
            pl.BlockSpec((_BN, _D), lambda p, g: ((1 - p) * g + p * (_G - 1), 0)),
            pl.BlockSpec((1, _BN), lambda p, g: (0, g)),
            pl.BlockSpec((_D, _H * _DP), lambda p, g: (0, 0)),
            pl.BlockSpec((_H, _DP), lambda p, g: (0, 0)),
        ],
        out_specs=pl.BlockSpec((_H, _BN), lambda p, g: (0, g)),
        out_shape=jax.ShapeDtypeStruct((_H, _N), jnp.float32),
        scratch_shapes=[
            pltpu.VMEM((_H, _N), jnp.float32),
            pltpu.VMEM((_H, _B), jnp.float32),
            pltpu.VMEM((_D, _H), jnp.float32),
        ],
        interpret=interpret,
    )


def kernel(inputs, segment_ids, lengths, W1, b1, W2, b2, W3, b3, Wr, br, W_k, W_q):
    seg_row = segment_ids.astype(jnp.int32).reshape(1, _N)
    out = _make_call()(inputs, seg_row, W_k, W_q)
    return out.reshape(_H, _N, 1)


# speedup vs baseline: 1.2864x; 1.2864x over previous
import math

import jax
import jax.numpy as jnp
from jax import lax
from jax.experimental import pallas as pl
from jax.experimental.pallas import tpu as pltpu

_N = 32768
_B = 16
_D = 128
_DP = 64
_H = 4
_BN = 8192
_G = _N // _BN
_SCALE = 1.0 / math.sqrt(float(_DP))


def _body(x_ref, seg_ref, wk_ref, wq_ref, out_ref, e_ref, stats_ref, u_ref):
    p = pl.program_id(0)
    g = pl.program_id(1)
    seg = jnp.broadcast_to(seg_ref[...], (_H, _BN))

    @pl.when((p == 0) & (g == 0))
    def _fold_u():
        wqt = jnp.transpose(wq_ref[...])
        tiled = jnp.concatenate([wqt] * _H, axis=0)
        r = lax.broadcasted_iota(jnp.int32, (_H * _DP, _H), 0)
        c = lax.broadcasted_iota(jnp.int32, (_H * _DP, _H), 1)
        wqbd = jnp.where(r // _DP == c, tiled, 0.0)
        u_ref[...] = lax.dot_general(wk_ref[...], wqbd,
                                     (((1,), (0,)), ((), ())),
                                     precision=lax.Precision.HIGHEST) * _SCALE

    @pl.when(p == 0)
    def _phase0():
        t = lax.dot_general(x_ref[...], u_ref[...], (((1,), (0,)), ((), ())))
        e = jnp.exp(jnp.minimum(jnp.transpose(t), 50.0))
        e_ref[:, pl.ds(g * _BN, _BN)] = e
        out_ref[...] = e
        ohT = (lax.broadcasted_iota(jnp.int32, (_B, _BN), 0)
               == seg_ref[...]).astype(jnp.float32)
        part = jnp.transpose(
            lax.dot_general(ohT, e, (((1,), (1,)), ((), ()))))

        @pl.when(g == 0)
        def _init():
            stats_ref[...] = part

        @pl.when(g != 0)
        def _acc():
            stats_ref[...] = stats_ref[...] + part

    @pl.when(p == 1)
    def _phase1():
        e = e_ref[:, pl.ds(g * _BN, _BN)]
        recip = 1.0 / jnp.maximum(stats_ref[...], 1e-30)
        ohT = (lax.broadcasted_iota(jnp.int32, (_B, _BN), 0)
               == seg_ref[...]).astype(jnp.float32)
        rg = lax.dot_general(recip, ohT, (((1,), (0,)), ((), ())),
                             precision=lax.Precision.HIGHEST)
        out_ref[...] = e * rg


def _make_call(interpret=False):
    return pl.pallas_call(
        _body,
        grid=(2, _G),
        in_specs=[
            pl.BlockSpec((_BN, _D), lambda p, g: ((1 - p) * g + p * (_G - 1), 0)),
            pl.BlockSpec((1, _BN), lambda p, g: (0, g)),
            pl.BlockSpec((_D, _H * _DP), lambda p, g: (0, 0)),
            pl.BlockSpec((_H, _DP), lambda p, g: (0, 0)),
        ],
        out_specs=pl.BlockSpec((_H, _BN), lambda p, g: (0, g)),
        out_shape=jax.ShapeDtypeStruct((_H, _N), jnp.float32),
        scratch_shapes=[
            pltpu.VMEM((_H, _N), jnp.float32),
            pltpu.VMEM((_H, _B), jnp.float32),
            pltpu.VMEM((_D, _H), jnp.float32),
        ],
        interpret=interpret,
    )


def kernel(inputs, segment_ids, lengths, W1, b1, W2, b2, W3, b3, Wr, br, W_k, W_q):
    seg_row = segment_ids.astype(jnp.int32).reshape(1, _N)
    out = _make_call()(inputs, seg_row, W_k, W_q)
    return out.reshape(_H, _N, 1)
